# Initial kernel scaffold; baseline (speedup 1.0000x reference)
import jax
import jax.numpy as jnp
from jax.experimental import pallas as pl


def kernel(xyz, features, neigh_idx, params):
    raise NotImplementedError("write your pallas kernel here")


# speedup vs baseline: 14.4635x; 14.4635x over previous
import functools
import jax
import jax.numpy as jnp
from jax import lax
from jax.experimental import pallas as pl
from jax.experimental.pallas import tpu as pltpu
from jax.experimental.pallas import tpu_sc as plsc

_N = 65536
_K = 16
_NB = 256
_GRID = _N // _NB
_NBK = _NB * _K


def _rsq(x, eps):
    return 1.0 / jnp.sqrt(x + eps)


def _lin_bn_consts(mu, M, Wt, g, b, eps):
    mean = mu @ Wt
    sec = jnp.sum((M @ Wt) * Wt, axis=0, keepdims=True)
    var = sec - mean * mean
    a = g * _rsq(var, eps)
    return a, b - a * mean


def _mom_bn_consts(s, sq, n, g, b, eps):
    mean = s / n
    var = sq / n - mean * mean
    a = g * _rsq(var, eps)
    return a, b - a * mean


def _sc_gather(table, idx):
    V, D = table.shape
    B = idx.shape[0]
    info = plsc.get_sparse_core_info()
    nw = info.num_cores * info.num_subcores
    b_per_w = B // nw
    ch = 4096
    n_ch = b_per_w // ch
    mesh = plsc.VectorSubcoreMesh(core_axis_name="c", subcore_axis_name="s")

    @functools.partial(
        pl.kernel, mesh=mesh,
        out_type=jax.ShapeDtypeStruct((B, D), jnp.float32),
        compiler_params=pltpu.CompilerParams(use_tc_tiling_on_sc=False),
        scratch_types=[
            pltpu.VMEM((ch,), jnp.int32),
            pltpu.VMEM((ch, D), jnp.float32),
            pltpu.SemaphoreType.DMA,
        ],
    )
    def k(table_hbm, idx_hbm, out_hbm, idx_v, rows_v, sem):
        wid = lax.axis_index("s") * info.num_cores + lax.axis_index("c")
        base = wid * b_per_w
        for c in range(n_ch):
            off = base + c * ch
            pltpu.sync_copy(idx_hbm.at[pl.ds(off, ch)], idx_v)
            pltpu.async_copy(table_hbm.at[idx_v], rows_v, sem).wait()
            pltpu.sync_copy(rows_v, out_hbm.at[pl.ds(off, ch)])

    return k(table, idx)


def _p1_body(feat, W0t, b0, f_out, s1, ssq, s2):
    i = pl.program_id(0)
    f = feat[...] @ W0t[...] + b0[...]
    f = jnp.where(f > 0, f, 0.01 * f)
    f_out[...] = f

    @pl.when(i == 0)
    def _():
        s1[...] = jnp.zeros_like(s1)
        ssq[...] = jnp.zeros_like(ssq)
        s2[...] = jnp.zeros_like(s2)

    s1[...] += jnp.sum(f, axis=0, keepdims=True)
    ssq[...] += jnp.sum(f * f, axis=0, keepdims=True)
    s2[...] += lax.dot_general(f, f, (((0,), (0,)), ((), ())))


def _p2_body(f_ref, xyz_ref, s1, ssq, s2, g0, b0, W1t, g1, b1,
             Wsct, gsc, bsc, t1_out, sc_out):
    n = float(_N)
    m0 = s1[...] / n
    var0 = ssq[...] / n - m0 * m0
    a0 = g0[...] * _rsq(var0, 1e-6)
    c0 = b0[...] - a0 * m0
    fn = a0 * f_ref[...] + c0

    M_f = s2[...] / n
    a0c = jnp.reshape(a0, (8, 1))
    m0c = jnp.reshape(m0, (8, 1))
    c0c = jnp.reshape(c0, (8, 1))
    mu_fn = a0 * m0 + c0
    M_fn = ((a0c * a0) * M_f + (a0c * m0c) * c0
            + c0c * (a0 * m0) + c0c * c0)

    a1, c1 = _lin_bn_consts(mu_fn, M_fn, W1t[...], g1[...], b1[...], 1e-5)
    f_pc = a1 * (fn @ W1t[...]) + c1
    f_pc = jnp.maximum(f_pc, 0.0)

    asc, csc = _lin_bn_consts(mu_fn, M_fn, Wsct[...], gsc[...], bsc[...], 1e-5)
    sc_out[...] = asc * (fn @ Wsct[...]) + csc

    t1_out[:, 0:3] = xyz_ref[...]
    t1_out[:, 3:8] = jnp.zeros((_NB, 5), jnp.float32)
    t1_out[:, 8:16] = f_pc


def _p3_body(g1_ref, xyz_ref, Wtd, Wtr, Wtt, Wtn, y_out, s1, ssq):
    i = pl.program_id(0)
    nb = g1_ref[:, 0:3]
    tile = jnp.reshape(
        jnp.broadcast_to(xyz_ref[...][:, None, :], (_NB, _K, 3)), (_NBK, 3))
    rel = tile - nb
    dist = jnp.sqrt(jnp.sum(rel * rel, axis=1, keepdims=True))
    y = dist * Wtd[...] + rel @ Wtr[...] + tile @ Wtt[...] + nb @ Wtn[...]
    y_out[...] = y

    @pl.when(i == 0)
    def _():
        s1[...] = jnp.zeros_like(s1)
        ssq[...] = jnp.zeros_like(ssq)

    s1[...] += jnp.sum(y, axis=0, keepdims=True)
    ssq[...] += jnp.sum(y * y, axis=0, keepdims=True)


def _p4_body(y_ref, g1_ref, sb1, sb1q, gbb1, bbb1, fcWt_nb, fcWt_fx,
             mlpWt_nb, mlpWt_fx, y_out, sfx, sfxfx, sa1, sa1q):
    i = pl.program_id(0)
    nk = float(_N * _K)
    a, c = _mom_bn_consts(sb1[...], sb1q[...], nk, gbb1[...], bbb1[...], 1e-5)
    fx = jnp.maximum(a * y_ref[...] + c, 0.0)
    fnb = g1_ref[:, 8:16]

    att = fnb @ fcWt_nb[...] + fx @ fcWt_fx[...]
    att3 = jnp.reshape(att, (_NB, _K, 16))
    m = jnp.max(att3, axis=1, keepdims=True)
    e = jnp.exp(att3 - m)
    sc3 = e / jnp.sum(e, axis=1, keepdims=True)

    fnb3 = jnp.reshape(fnb, (_NB, _K, 8))
    fx3 = jnp.reshape(fx, (_NB, _K, 8))
    agg_nb = jnp.sum(fnb3 * sc3[:, :, 0:8], axis=1)
    agg_fx = jnp.sum(fx3 * sc3[:, :, 8:16], axis=1)
    y = agg_nb @ mlpWt_nb[...] + agg_fx @ mlpWt_fx[...]
    y_out[...] = y

    @pl.when(i == 0)
    def _():
        sfx[...] = jnp.zeros_like(sfx)
        sfxfx[...] = jnp.zeros_like(sfxfx)
        sa1[...] = jnp.zeros_like(sa1)
        sa1q[...] = jnp.zeros_like(sa1q)

    sfx[...] += jnp.sum(fx, axis=0, keepdims=True)
    sfxfx[...] += lax.dot_general(fx, fx, (((0,), (0,)), ((), ())))
    sa1[...] += jnp.sum(y, axis=0, keepdims=True)
    sa1q[...] += jnp.sum(y * y, axis=0, keepdims=True)


def _p5_body(y_ref, sa1, sa1q, gap1, bap1, t2_out):
    n = float(_N)
    a, c = _mom_bn_consts(sa1[...], sa1q[...], n, gap1[...], bap1[...], 1e-5)
    fagg = jnp.maximum(a * y_ref[...] + c, 0.0)
    t2_out[:, 0:8] = fagg
    t2_out[:, 8:16] = jnp.zeros((_NB, 8), jnp.float32)


def _p6_body(y_ref, g2_ref, sb1, sb1q, sfx, sfxfx, gbb1, bbb1, W2t, gbb2,
             bbb2, fc2Wt_nb, fc2Wt_fx, mlp2Wt_nb, mlp2Wt_fx,
             y_out, sa2, sa2q):
    i = pl.program_id(0)
    nk = float(_N * _K)
    a, c = _mom_bn_consts(sb1[...], sb1q[...], nk, gbb1[...], bbb1[...], 1e-5)
    fx = jnp.maximum(a * y_ref[...] + c, 0.0)

    mu_fx = sfx[...] / nk
    M_fx = sfxfx[...] / nk
    a2, c2 = _lin_bn_consts(mu_fx, M_fx, W2t[...], gbb2[...], bbb2[...], 1e-5)
    fx2 = jnp.maximum(a2 * (fx @ W2t[...]) + c2, 0.0)

    fnb2 = g2_ref[:, 0:8]
    att = fnb2 @ fc2Wt_nb[...] + fx2 @ fc2Wt_fx[...]
    att3 = jnp.reshape(att, (_NB, _K, 16))
    m = jnp.max(att3, axis=1, keepdims=True)
    e = jnp.exp(att3 - m)
    sc3 = e / jnp.sum(e, axis=1, keepdims=True)

    fnb3 = jnp.reshape(fnb2, (_NB, _K, 8))
    fx3 = jnp.reshape(fx2, (_NB, _K, 8))
    agg_nb = jnp.sum(fnb3 * sc3[:, :, 0:8], axis=1)
    agg_fx = jnp.sum(fx3 * sc3[:, :, 8:16], axis=1)
    y = agg_nb @ mlp2Wt_nb[...] + agg_fx @ mlp2Wt_fx[...]
    y_out[...] = y

    @pl.when(i == 0)
    def _():
        sa2[...] = jnp.zeros_like(sa2)
        sa2q[...] = jnp.zeros_like(sa2q)

    sa2[...] += jnp.sum(y, axis=0, keepdims=True)
    sa2q[...] += jnp.sum(y * y, axis=0, keepdims=True)


def _p7_body(y_ref, sa2, sa2q, gap2, bap2, drbWt, y_out, sfl, sflfl):
    i = pl.program_id(0)
    n = float(_N)
    a, c = _mom_bn_consts(sa2[...], sa2q[...], n, gap2[...], bap2[...], 1e-5)
    fl = jnp.maximum(a * y_ref[...] + c, 0.0)
    y_out[...] = fl @ drbWt[...]

    @pl.when(i == 0)
    def _():
        sfl[...] = jnp.zeros_like(sfl)
        sflfl[...] = jnp.zeros_like(sflfl)

    sfl[...] += jnp.sum(fl, axis=0, keepdims=True)
    sflfl[...] += lax.dot_general(fl, fl, (((0,), (0,)), ((), ())))


def _p8_body(y_ref, sc_ref, sfl, sflfl, drbWt, gdrb, bdrb, out):
    n = float(_N)
    mu = sfl[...] / n
    M = sflfl[...] / n
    a, c = _lin_bn_consts(mu, M, drbWt[...], gdrb[...], bdrb[...], 1e-5)
    z = a * y_ref[...] + c + sc_ref[...]
    out[...] = jnp.where(z > 0, z, 0.2 * z)


def _blk(c):
    return pl.BlockSpec((_NB, c), lambda i: (i, 0))


def _blkk(c):
    return pl.BlockSpec((_NBK, c), lambda i: (i, 0))


def _full(shape):
    nd = len(shape)
    return pl.BlockSpec(shape, lambda i: (0,) * nd)


def _row(x):
    return jnp.asarray(x, jnp.float32).reshape(1, -1)


def kernel(xyz, features, neigh_idx, params):
    p = params
    xyz2 = xyz[0]
    feat2 = features[0]
    idx_flat = neigh_idx[0].astype(jnp.int32).reshape(-1)

    f32 = jnp.float32
    sd = jax.ShapeDtypeStruct

    f_arr, s1, ssq, s2 = pl.pallas_call(
        _p1_body,
        grid=(_GRID,),
        in_specs=[_blk(6), _full((6, 8)), _full((1, 8))],
        out_specs=[_blk(8), _full((1, 8)), _full((1, 8)), _full((8, 8))],
        out_shape=[sd((_N, 8), f32), sd((1, 8), f32), sd((1, 8), f32),
                   sd((8, 8), f32)],
    )(feat2, p['fc0_W'].T, _row(p['fc0_b']))

    t1, sc_out = pl.pallas_call(
        _p2_body,
        grid=(_GRID,),
        in_specs=[_blk(8), _blk(3), _full((1, 8)), _full((1, 8)),
                  _full((8, 8)), _full((1, 8)), _full((1, 8)),
                  _full((8, 8)), _full((1, 8)), _full((1, 8)),
                  _full((8, 32)), _full((1, 32)), _full((1, 32))],
        out_specs=[_blk(16), _blk(32)],
        out_shape=[sd((_N, 16), f32), sd((_N, 32), f32)],
    )(f_arr, xyz2, s1, ssq, s2, _row(p['bn0_g']), _row(p['bn0_b']),
      p['drb_mlp1_W'].T, _row(p['drb_mlp1_g']), _row(p['drb_mlp1_b']),
      p['sc_W'].T, _row(p['sc_g']), _row(p['sc_b']))

    g1 = _sc_gather(t1, idx_flat)

    bbWt = p['bb_mlp1_W'].T
    y_bb1, sb1, sb1q = pl.pallas_call(
        _p3_body,
        grid=(_GRID,),
        in_specs=[_blkk(16), _blk(3), _full((1, 8)), _full((3, 8)),
                  _full((3, 8)), _full((3, 8))],
        out_specs=[_blkk(8), _full((1, 8)), _full((1, 8))],
        out_shape=[sd((_N * _K, 8), f32), sd((1, 8), f32), sd((1, 8), f32)],
    )(g1, xyz2, bbWt[0:1], bbWt[1:4], bbWt[4:7], bbWt[7:10])

    fcW = p['ap1_fc_W']
    mlpW = p['ap1_mlp_W']
    y_ap1, sfx, sfxfx, sa1, sa1q = pl.pallas_call(
        _p4_body,
        grid=(_GRID,),
        in_specs=[_blkk(8), _blkk(16), _full((1, 8)), _full((1, 8)),
                  _full((1, 8)), _full((1, 8)), _full((8, 16)),
                  _full((8, 16)), _full((8, 8)), _full((8, 8))],
        out_specs=[_blk(8), _full((1, 8)), _full((8, 8)), _full((1, 8)),
                   _full((1, 8))],
        out_shape=[sd((_N, 8), f32), sd((1, 8), f32), sd((8, 8), f32),
                   sd((1, 8), f32), sd((1, 8), f32)],
    )(y_bb1, g1, sb1, sb1q, _row(p['bb_mlp1_g']), _row(p['bb_mlp1_b']),
      fcW[:, 0:8].T, fcW[:, 8:16].T, mlpW[:, 0:8].T, mlpW[:, 8:16].T)

    t2 = pl.pallas_call(
        _p5_body,
        grid=(_GRID,),
        in_specs=[_blk(8), _full((1, 8)), _full((1, 8)), _full((1, 8)),
                  _full((1, 8))],
        out_specs=_blk(16),
        out_shape=sd((_N, 16), f32),
    )(y_ap1, sa1, sa1q, _row(p['ap1_g']), _row(p['ap1_b']))

    g2 = _sc_gather(t2, idx_flat)

    fc2W = p['ap2_fc_W']
    mlp2W = p['ap2_mlp_W']
    y_ap2, sa2, sa2q = pl.pallas_call(
        _p6_body,
        grid=(_GRID,),
        in_specs=[_blkk(8), _blkk(16), _full((1, 8)), _full((1, 8)),
                  _full((1, 8)), _full((8, 8)), _full((1, 8)),
                  _full((1, 8)), _full((8, 8)), _full((1, 8)),
                  _full((1, 8)), _full((8, 16)), _full((8, 16)),
                  _full((8, 16)), _full((8, 16))],
        out_specs=[_blk(16), _full((1, 16)), _full((1, 16))],
        out_shape=[sd((_N, 16), f32), sd((1, 16), f32), sd((1, 16), f32)],
    )(y_bb1, g2, sb1, sb1q, sfx, sfxfx, _row(p['bb_mlp1_g']),
      _row(p['bb_mlp1_b']), p['bb_mlp2_W'].T, _row(p['bb_mlp2_g']),
      _row(p['bb_mlp2_b']), fc2W[:, 0:8].T, fc2W[:, 8:16].T,
      mlp2W[:, 0:8].T, mlp2W[:, 8:16].T)

    y_drb, sfl, sflfl = pl.pallas_call(
        _p7_body,
        grid=(_GRID,),
        in_specs=[_blk(16), _full((1, 16)), _full((1, 16)),
                  _full((1, 16)), _full((1, 16)), _full((16, 32))],
        out_specs=[_blk(32), _full((1, 16)), _full((16, 16))],
        out_shape=[sd((_N, 32), f32), sd((1, 16), f32), sd((16, 16), f32)],
    )(y_ap2, sa2, sa2q, _row(p['ap2_g']), _row(p['ap2_b']),
      p['drb_mlp2_W'].T)

    out = pl.pallas_call(
        _p8_body,
        grid=(_GRID,),
        in_specs=[_blk(32), _blk(32), _full((1, 16)), _full((16, 16)),
                  _full((16, 32)), _full((1, 32)), _full((1, 32))],
        out_specs=_blk(32),
        out_shape=sd((_N, 32), f32),
    )(y_drb, sc_out, sfl, sflfl, p['drb_mlp2_W'].T,
      _row(p['drb_mlp2_g']), _row(p['drb_mlp2_b']))

    return out.T[None, :, :, None]


# speedup vs baseline: 18.1826x; 1.2571x over previous
import functools
import jax
import jax.numpy as jnp
from jax import lax
from jax.experimental import pallas as pl
from jax.experimental.pallas import tpu as pltpu
from jax.experimental.pallas import tpu_sc as plsc

_N = 65536
_K = 16
_NB = 512
_GRID = _N // _NB
_NBK = _NB * _K
_NBP = 4096
_GRIDP = _N // _NBP


def _rsq(x, eps):
    return 1.0 / jnp.sqrt(x + eps)


def _lin_bn_consts(mu, M, Wt, g, b, eps):
    mean = mu @ Wt
    sec = jnp.sum((M @ Wt) * Wt, axis=0, keepdims=True)
    var = sec - mean * mean
    a = g * _rsq(var, eps)
    return a, b - a * mean


def _mom_bn_consts(s, sq, n, g, b, eps):
    mean = s / n
    var = sq / n - mean * mean
    a = g * _rsq(var, eps)
    return a, b - a * mean


def _sc_gather(table, idx):
    V, D = table.shape
    B = idx.shape[0]
    info = plsc.get_sparse_core_info()
    nw = info.num_cores * info.num_subcores
    b_per_w = B // nw
    ch = 4096
    n_ch = b_per_w // ch
    mesh = plsc.VectorSubcoreMesh(core_axis_name="c", subcore_axis_name="s")

    @functools.partial(
        pl.kernel, mesh=mesh,
        out_type=jax.ShapeDtypeStruct((B, D), jnp.float32),
        compiler_params=pltpu.CompilerParams(use_tc_tiling_on_sc=False),
        scratch_types=[
            pltpu.VMEM((ch,), jnp.int32),
            pltpu.VMEM((ch, D), jnp.float32),
            pltpu.SemaphoreType.DMA,
        ],
    )
    def k(table_hbm, idx_hbm, out_hbm, idx_v, rows_v, sem):
        wid = lax.axis_index("s") * info.num_cores + lax.axis_index("c")
        base = wid * b_per_w
        for c in range(n_ch):
            off = base + c * ch
            pltpu.sync_copy(idx_hbm.at[pl.ds(off, ch)], idx_v)
            pltpu.async_copy(table_hbm.at[idx_v], rows_v, sem).wait()
            pltpu.sync_copy(rows_v, out_hbm.at[pl.ds(off, ch)])

    return k(table, idx)


def _p1_body(feat, W0t, b0, f_out, s1, ssq, s2):
    i = pl.program_id(0)
    f = feat[...] @ W0t[...] + b0[...]
    f = jnp.where(f > 0, f, 0.01 * f)
    f_out[...] = f

    @pl.when(i == 0)
    def _():
        s1[...] = jnp.zeros_like(s1)
        ssq[...] = jnp.zeros_like(ssq)
        s2[...] = jnp.zeros_like(s2)

    s1[...] += jnp.sum(f, axis=0, keepdims=True)
    ssq[...] += jnp.sum(f * f, axis=0, keepdims=True)
    s2[...] += lax.dot_general(f, f, (((0,), (0,)), ((), ())))


def _p2_body(f_ref, xyz_ref, s1, ssq, s2, g0, b0, W1t, g1, b1,
             Wsct, gsc, bsc, t1_out, sc_out):
    n = float(_N)
    m0 = s1[...] / n
    var0 = ssq[...] / n - m0 * m0
    a0 = g0[...] * _rsq(var0, 1e-6)
    c0 = b0[...] - a0 * m0
    fn = a0 * f_ref[...] + c0

    M_f = s2[...] / n
    a0c = jnp.reshape(a0, (8, 1))
    m0c = jnp.reshape(m0, (8, 1))
    c0c = jnp.reshape(c0, (8, 1))
    mu_fn = a0 * m0 + c0
    M_fn = ((a0c * a0) * M_f + (a0c * m0c) * c0
            + c0c * (a0 * m0) + c0c * c0)

    a1, c1 = _lin_bn_consts(mu_fn, M_fn, W1t[...], g1[...], b1[...], 1e-5)
    f_pc = a1 * (fn @ W1t[...]) + c1
    f_pc = jnp.maximum(f_pc, 0.0)

    asc, csc = _lin_bn_consts(mu_fn, M_fn, Wsct[...], gsc[...], bsc[...], 1e-5)
    sc_out[...] = asc * (fn @ Wsct[...]) + csc

    t1_out[:, 0:3] = xyz_ref[...]
    t1_out[:, 3:8] = jnp.zeros((_NBP, 5), jnp.float32)
    t1_out[:, 8:16] = f_pc


def _p3_body(g1_ref, xyz_ref, Wtd, Wtr, Wtt, Wtn, y_out, s1, ssq):
    i = pl.program_id(0)
    nb = g1_ref[:, 0:3]
    tile = jnp.reshape(
        jnp.broadcast_to(xyz_ref[...][:, None, :], (_NB, _K, 3)), (_NBK, 3))
    rel = tile - nb
    dist = jnp.sqrt(jnp.sum(rel * rel, axis=1, keepdims=True))
    y = dist * Wtd[...] + rel @ Wtr[...] + tile @ Wtt[...] + nb @ Wtn[...]
    y_out[...] = y

    @pl.when(i == 0)
    def _():
        s1[...] = jnp.zeros_like(s1)
        ssq[...] = jnp.zeros_like(ssq)

    s1[...] += jnp.sum(y, axis=0, keepdims=True)
    ssq[...] += jnp.sum(y * y, axis=0, keepdims=True)


def _p4_body(y_ref, g1_ref, sb1, sb1q, gbb1, bbb1, fcWt_nb, fcWt_fx,
             mlpWt_nb, mlpWt_fx, y_out, sfx, sfxfx, sa1, sa1q):
    i = pl.program_id(0)
    nk = float(_N * _K)
    a, c = _mom_bn_consts(sb1[...], sb1q[...], nk, gbb1[...], bbb1[...], 1e-5)
    fx = jnp.maximum(a * y_ref[...] + c, 0.0)
    fnb = g1_ref[:, 8:16]

    att = fnb @ fcWt_nb[...] + fx @ fcWt_fx[...]
    att3 = jnp.reshape(att, (_NB, _K, 16))
    m = jnp.max(att3, axis=1, keepdims=True)
    e = jnp.exp(att3 - m)
    sc3 = e / jnp.sum(e, axis=1, keepdims=True)

    fnb3 = jnp.reshape(fnb, (_NB, _K, 8))
    fx3 = jnp.reshape(fx, (_NB, _K, 8))
    agg_nb = jnp.sum(fnb3 * sc3[:, :, 0:8], axis=1)
    agg_fx = jnp.sum(fx3 * sc3[:, :, 8:16], axis=1)
    y = agg_nb @ mlpWt_nb[...] + agg_fx @ mlpWt_fx[...]
    y_out[...] = y

    @pl.when(i == 0)
    def _():
        sfx[...] = jnp.zeros_like(sfx)
        sfxfx[...] = jnp.zeros_like(sfxfx)
        sa1[...] = jnp.zeros_like(sa1)
        sa1q[...] = jnp.zeros_like(sa1q)

    sfx[...] += jnp.sum(fx, axis=0, keepdims=True)
    sfxfx[...] += lax.dot_general(fx, fx, (((0,), (0,)), ((), ())))
    sa1[...] += jnp.sum(y, axis=0, keepdims=True)
    sa1q[...] += jnp.sum(y * y, axis=0, keepdims=True)


def _p5_body(y_ref, sa1, sa1q, gap1, bap1, t2_out):
    n = float(_N)
    a, c = _mom_bn_consts(sa1[...], sa1q[...], n, gap1[...], bap1[...], 1e-5)
    fagg = jnp.maximum(a * y_ref[...] + c, 0.0)
    t2_out[:, 0:8] = fagg
    t2_out[:, 8:16] = jnp.zeros((_NBP, 8), jnp.float32)


def _p6_body(y_ref, g2_ref, sb1, sb1q, sfx, sfxfx, gbb1, bbb1, W2t, gbb2,
             bbb2, fc2Wt_nb, fc2Wt_fx, mlp2Wt_nb, mlp2Wt_fx,
             y_out, sa2, sa2q):
    i = pl.program_id(0)
    nk = float(_N * _K)
    a, c = _mom_bn_consts(sb1[...], sb1q[...], nk, gbb1[...], bbb1[...], 1e-5)
    fx = jnp.maximum(a * y_ref[...] + c, 0.0)

    mu_fx = sfx[...] / nk
    M_fx = sfxfx[...] / nk
    a2, c2 = _lin_bn_consts(mu_fx, M_fx, W2t[...], gbb2[...], bbb2[...], 1e-5)
    fx2 = jnp.maximum(a2 * (fx @ W2t[...]) + c2, 0.0)

    fnb2 = g2_ref[:, 0:8]
    att = fnb2 @ fc2Wt_nb[...] + fx2 @ fc2Wt_fx[...]
    att3 = jnp.reshape(att, (_NB, _K, 16))
    m = jnp.max(att3, axis=1, keepdims=True)
    e = jnp.exp(att3 - m)
    sc3 = e / jnp.sum(e, axis=1, keepdims=True)

    fnb3 = jnp.reshape(fnb2, (_NB, _K, 8))
    fx3 = jnp.reshape(fx2, (_NB, _K, 8))
    agg_nb = jnp.sum(fnb3 * sc3[:, :, 0:8], axis=1)
    agg_fx = jnp.sum(fx3 * sc3[:, :, 8:16], axis=1)
    y = agg_nb @ mlp2Wt_nb[...] + agg_fx @ mlp2Wt_fx[...]
    y_out[...] = y

    @pl.when(i == 0)
    def _():
        sa2[...] = jnp.zeros_like(sa2)
        sa2q[...] = jnp.zeros_like(sa2q)

    sa2[...] += jnp.sum(y, axis=0, keepdims=True)
    sa2q[...] += jnp.sum(y * y, axis=0, keepdims=True)


def _p7_body(y_ref, sa2, sa2q, gap2, bap2, drbWt, y_out, sfl, sflfl):
    i = pl.program_id(0)
    n = float(_N)
    a, c = _mom_bn_consts(sa2[...], sa2q[...], n, gap2[...], bap2[...], 1e-5)
    fl = jnp.maximum(a * y_ref[...] + c, 0.0)
    y_out[...] = fl @ drbWt[...]

    @pl.when(i == 0)
    def _():
        sfl[...] = jnp.zeros_like(sfl)
        sflfl[...] = jnp.zeros_like(sflfl)

    sfl[...] += jnp.sum(fl, axis=0, keepdims=True)
    sflfl[...] += lax.dot_general(fl, fl, (((0,), (0,)), ((), ())))


def _p8_body(y_ref, sc_ref, sfl, sflfl, drbWt, gdrb, bdrb, out):
    n = float(_N)
    mu = sfl[...] / n
    M = sflfl[...] / n
    a, c = _lin_bn_consts(mu, M, drbWt[...], gdrb[...], bdrb[...], 1e-5)
    z = a * y_ref[...] + c + sc_ref[...]
    out[...] = jnp.where(z > 0, z, 0.2 * z)


def _blk(c):
    return pl.BlockSpec((_NB, c), lambda i: (i, 0))


def _blkp(c):
    return pl.BlockSpec((_NBP, c), lambda i: (i, 0))


def _blkk(c):
    return pl.BlockSpec((_NBK, c), lambda i: (i, 0))


def _full(shape):
    nd = len(shape)
    return pl.BlockSpec(shape, lambda i: (0,) * nd)


def _row(x):
    return jnp.asarray(x, jnp.float32).reshape(1, -1)


def kernel(xyz, features, neigh_idx, params):
    p = params
    xyz2 = xyz[0]
    feat2 = features[0]
    idx_flat = neigh_idx[0].astype(jnp.int32).reshape(-1)

    f32 = jnp.float32
    sd = jax.ShapeDtypeStruct

    f_arr, s1, ssq, s2 = pl.pallas_call(
        _p1_body,
        grid=(_GRIDP,),
        in_specs=[_blkp(6), _full((6, 8)), _full((1, 8))],
        out_specs=[_blkp(8), _full((1, 8)), _full((1, 8)), _full((8, 8))],
        out_shape=[sd((_N, 8), f32), sd((1, 8), f32), sd((1, 8), f32),
                   sd((8, 8), f32)],
    )(feat2, p['fc0_W'].T, _row(p['fc0_b']))

    t1, sc_out = pl.pallas_call(
        _p2_body,
        grid=(_GRIDP,),
        in_specs=[_blkp(8), _blkp(3), _full((1, 8)), _full((1, 8)),
                  _full((8, 8)), _full((1, 8)), _full((1, 8)),
                  _full((8, 8)), _full((1, 8)), _full((1, 8)),
                  _full((8, 32)), _full((1, 32)), _full((1, 32))],
        out_specs=[_blkp(16), _blkp(32)],
        out_shape=[sd((_N, 16), f32), sd((_N, 32), f32)],
    )(f_arr, xyz2, s1, ssq, s2, _row(p['bn0_g']), _row(p['bn0_b']),
      p['drb_mlp1_W'].T, _row(p['drb_mlp1_g']), _row(p['drb_mlp1_b']),
      p['sc_W'].T, _row(p['sc_g']), _row(p['sc_b']))

    g1 = _sc_gather(t1, idx_flat)

    bbWt = p['bb_mlp1_W'].T
    y_bb1, sb1, sb1q = pl.pallas_call(
        _p3_body,
        grid=(_GRID,),
        in_specs=[_blkk(16), _blk(3), _full((1, 8)), _full((3, 8)),
                  _full((3, 8)), _full((3, 8))],
        out_specs=[_blkk(8), _full((1, 8)), _full((1, 8))],
        out_shape=[sd((_N * _K, 8), f32), sd((1, 8), f32), sd((1, 8), f32)],
    )(g1, xyz2, bbWt[0:1], bbWt[1:4], bbWt[4:7], bbWt[7:10])

    fcW = p['ap1_fc_W']
    mlpW = p['ap1_mlp_W']
    y_ap1, sfx, sfxfx, sa1, sa1q = pl.pallas_call(
        _p4_body,
        grid=(_GRID,),
        in_specs=[_blkk(8), _blkk(16), _full((1, 8)), _full((1, 8)),
                  _full((1, 8)), _full((1, 8)), _full((8, 16)),
                  _full((8, 16)), _full((8, 8)), _full((8, 8))],
        out_specs=[_blk(8), _full((1, 8)), _full((8, 8)), _full((1, 8)),
                   _full((1, 8))],
        out_shape=[sd((_N, 8), f32), sd((1, 8), f32), sd((8, 8), f32),
                   sd((1, 8), f32), sd((1, 8), f32)],
    )(y_bb1, g1, sb1, sb1q, _row(p['bb_mlp1_g']), _row(p['bb_mlp1_b']),
      fcW[:, 0:8].T, fcW[:, 8:16].T, mlpW[:, 0:8].T, mlpW[:, 8:16].T)

    t2 = pl.pallas_call(
        _p5_body,
        grid=(_GRIDP,),
        in_specs=[_blkp(8), _full((1, 8)), _full((1, 8)), _full((1, 8)),
                  _full((1, 8))],
        out_specs=_blkp(16),
        out_shape=sd((_N, 16), f32),
    )(y_ap1, sa1, sa1q, _row(p['ap1_g']), _row(p['ap1_b']))

    g2 = _sc_gather(t2, idx_flat)

    fc2W = p['ap2_fc_W']
    mlp2W = p['ap2_mlp_W']
    y_ap2, sa2, sa2q = pl.pallas_call(
        _p6_body,
        grid=(_GRID,),
        in_specs=[_blkk(8), _blkk(16), _full((1, 8)), _full((1, 8)),
                  _full((1, 8)), _full((8, 8)), _full((1, 8)),
                  _full((1, 8)), _full((8, 8)), _full((1, 8)),
                  _full((1, 8)), _full((8, 16)), _full((8, 16)),
                  _full((8, 16)), _full((8, 16))],
        out_specs=[_blk(16), _full((1, 16)), _full((1, 16))],
        out_shape=[sd((_N, 16), f32), sd((1, 16), f32), sd((1, 16), f32)],
    )(y_bb1, g2, sb1, sb1q, sfx, sfxfx, _row(p['bb_mlp1_g']),
      _row(p['bb_mlp1_b']), p['bb_mlp2_W'].T, _row(p['bb_mlp2_g']),
      _row(p['bb_mlp2_b']), fc2W[:, 0:8].T, fc2W[:, 8:16].T,
      mlp2W[:, 0:8].T, mlp2W[:, 8:16].T)

    y_drb, sfl, sflfl = pl.pallas_call(
        _p7_body,
        grid=(_GRIDP,),
        in_specs=[_blkp(16), _full((1, 16)), _full((1, 16)),
                  _full((1, 16)), _full((1, 16)), _full((16, 32))],
        out_specs=[_blkp(32), _full((1, 16)), _full((16, 16))],
        out_shape=[sd((_N, 32), f32), sd((1, 16), f32), sd((16, 16), f32)],
    )(y_ap2, sa2, sa2q, _row(p['ap2_g']), _row(p['ap2_b']),
      p['drb_mlp2_W'].T)

    out = pl.pallas_call(
        _p8_body,
        grid=(_GRIDP,),
        in_specs=[_blkp(32), _blkp(32), _full((1, 16)), _full((16, 16)),
                  _full((16, 32)), _full((1, 32)), _full((1, 32))],
        out_specs=_blkp(32),
        out_shape=sd((_N, 32), f32),
    )(y_drb, sc_out, sfl, sflfl, p['drb_mlp2_W'].T,
      _row(p['drb_mlp2_g']), _row(p['drb_mlp2_b']))

    return out.T[None, :, :, None]
